# baseline (device time: 15983 ns/iter reference)
import jax
import jax.numpy as jnp
from jax import lax
from jax.experimental import pallas as pl
from jax.experimental.pallas import tpu as pltpu

N_DEV = 4
BLK = 8
T_CORR = 64


def kernel(x, A, B, C):
    b, s, d = x.shape
    n = A.shape[-1]
    f32 = jnp.float32

    def body(x_ref, a_ref, b_ref, c_ref, out_ref, comm_ref, send_sem, recv_sem):
        my = lax.axis_index("i")
        left = lax.rem(my + N_DEV - 1, N_DEV)
        right = lax.rem(my + 1, N_DEV)

        barrier = pltpu.get_barrier_semaphore()
        for nbr in (left, right):
            pl.semaphore_signal(barrier, inc=1, device_id=(nbr,),
                                device_id_type=pl.DeviceIdType.MESH)
        pl.semaphore_wait(barrier, 2)

        dA = jnp.exp(a_ref[:, :].astype(f32))
        dAT = dA.T[None]

        def step(h, xt, Bt, Ct):
            h = h * dAT + xt[:, None, :] * Bt[:, :, None]
            yt = jnp.sum(h * Ct[:, :, None], axis=1)
            return h, yt

        def blk(i, h):
            t0 = i * BLK
            xblk = x_ref[:, pl.ds(t0, BLK), :].astype(f32)
            Bblk = b_ref[:, pl.ds(t0, BLK), :].astype(f32)
            Cblk = c_ref[:, pl.ds(t0, BLK), :].astype(f32)
            ys = []
            for j in range(BLK):
                h, yt = step(h, xblk[:, j, :], Bblk[:, j, :], Cblk[:, j, :])
                ys.append(yt)
            out_ref[:, pl.ds(t0, BLK), :] = jnp.stack(ys, axis=1)
            return h

        h_fin = lax.fori_loop(0, s // BLK, blk, jnp.zeros((b, n, d), f32))

        comm_ref[0] = h_fin
        rdma = pltpu.make_async_remote_copy(
            src_ref=comm_ref.at[0],
            dst_ref=comm_ref.at[1],
            send_sem=send_sem,
            recv_sem=recv_sem,
            device_id=(right,),
            device_id_type=pl.DeviceIdType.MESH,
        )
        rdma.start()
        rdma.wait_recv()

        @pl.when(my != 0)
        def _():
            def corr_blk(i, g):
                t0 = i * BLK
                Cblk = c_ref[:, pl.ds(t0, BLK), :].astype(f32)
                yblk = out_ref[:, pl.ds(t0, BLK), :]
                ys = []
                for j in range(BLK):
                    g = g * dAT
                    Ct = Cblk[:, j, :]
                    ys.append(yblk[:, j, :] + jnp.sum(g * Ct[:, :, None], axis=1))
                out_ref[:, pl.ds(t0, BLK), :] = jnp.stack(ys, axis=1)
                return g

            lax.fori_loop(0, T_CORR // BLK, corr_blk, comm_ref[1])

        rdma.wait_send()

    return pl.pallas_call(
        body,
        out_shape=jax.ShapeDtypeStruct((b, s, d), f32),
        in_specs=[pl.BlockSpec(memory_space=pltpu.VMEM)] * 4,
        out_specs=pl.BlockSpec(memory_space=pltpu.VMEM),
        scratch_shapes=[
            pltpu.VMEM((2, b, n, d), f32),
            pltpu.SemaphoreType.DMA,
            pltpu.SemaphoreType.DMA,
        ],
        compiler_params=pltpu.CompilerParams(collective_id=0),
    )(x, A, B, C)


# device time: 15420 ns/iter; 1.0365x vs baseline; 1.0365x over previous
import jax
import jax.numpy as jnp
from jax import lax
from jax.experimental import pallas as pl
from jax.experimental.pallas import tpu as pltpu

N_DEV = 4
BLK = 8
T_CORR = 32


def kernel(x, A, B, C):
    b, s, d = x.shape
    n = A.shape[-1]
    f32 = jnp.float32
    bf = jnp.bfloat16

    def body(x_ref, a_ref, b_ref, c_ref, out_ref, xbf_ref, comm_ref,
             send_sem, recv_sem):
        my = lax.axis_index("i")
        left = lax.rem(my + N_DEV - 1, N_DEV)
        right = lax.rem(my + 1, N_DEV)

        barrier = pltpu.get_barrier_semaphore()
        for nbr in (left, right):
            pl.semaphore_signal(barrier, inc=1, device_id=(nbr,),
                                device_id_type=pl.DeviceIdType.MESH)
        pl.semaphore_wait(barrier, 2)

        dA = jnp.exp(a_ref[:, :].astype(f32))
        dAT = dA.T[None].astype(bf)

        xbf_ref[...] = x_ref[...].astype(bf)

        def step(h, xt, Bt, Ct):
            h = h * dAT + xt[:, None, :] * Bt[:, :, None]
            yt = jnp.sum(h * Ct[:, :, None], axis=1)
            return h, yt

        def blk(i, h):
            t0 = i * BLK
            xblk = xbf_ref[:, pl.ds(t0, BLK), :]
            Bblk = b_ref[:, pl.ds(t0, BLK), :].astype(bf)
            Cblk = c_ref[:, pl.ds(t0, BLK), :].astype(bf)
            ys = []
            for j in range(BLK):
                h, yt = step(h, xblk[:, j, :], Bblk[:, j, :], Cblk[:, j, :])
                ys.append(yt)
            out_ref[:, pl.ds(t0, BLK), :] = jnp.stack(ys, axis=1).astype(f32)
            return h

        h_fin = lax.fori_loop(0, s // BLK, blk, jnp.zeros((b, n, d), bf))

        comm_ref[0] = h_fin
        rdma = pltpu.make_async_remote_copy(
            src_ref=comm_ref.at[0],
            dst_ref=comm_ref.at[1],
            send_sem=send_sem,
            recv_sem=recv_sem,
            device_id=(right,),
            device_id_type=pl.DeviceIdType.MESH,
        )
        rdma.start()
        rdma.wait_recv()

        @pl.when(my != 0)
        def _():
            def corr_blk(i, g):
                t0 = i * BLK
                Cblk = c_ref[:, pl.ds(t0, BLK), :].astype(bf)
                yblk = out_ref[:, pl.ds(t0, BLK), :]
                ys = []
                for j in range(BLK):
                    g = g * dAT
                    Ct = Cblk[:, j, :]
                    corr = jnp.sum(g * Ct[:, :, None], axis=1).astype(f32)
                    ys.append(yblk[:, j, :] + corr)
                out_ref[:, pl.ds(t0, BLK), :] = jnp.stack(ys, axis=1)
                return g

            lax.fori_loop(0, T_CORR // BLK, corr_blk, comm_ref[1])

        rdma.wait_send()

    return pl.pallas_call(
        body,
        out_shape=jax.ShapeDtypeStruct((b, s, d), f32),
        in_specs=[pl.BlockSpec(memory_space=pltpu.VMEM)] * 4,
        out_specs=pl.BlockSpec(memory_space=pltpu.VMEM),
        scratch_shapes=[
            pltpu.VMEM((b, s, d), bf),
            pltpu.VMEM((2, b, n, d), bf),
            pltpu.SemaphoreType.DMA,
            pltpu.SemaphoreType.DMA,
        ],
        compiler_params=pltpu.CompilerParams(collective_id=0),
    )(x, A, B, C)


# device time: 13356 ns/iter; 1.1967x vs baseline; 1.1545x over previous
import jax
import jax.numpy as jnp
from jax import lax
from jax.experimental import pallas as pl
from jax.experimental.pallas import tpu as pltpu

N_DEV = 4
BLK = 8
T_CORR = 32


def kernel(x, A, B, C):
    b, s, d = x.shape
    n = A.shape[-1]
    f32 = jnp.float32
    bf = jnp.bfloat16

    def body(x_ref, a_ref, b_ref, c_ref, out_ref, u_ref, comm_ref,
             send_sem, recv_sem):
        my = lax.axis_index("i")
        left = lax.rem(my + N_DEV - 1, N_DEV)
        right = lax.rem(my + 1, N_DEV)

        barrier = pltpu.get_barrier_semaphore()
        for nbr in (left, right):
            pl.semaphore_signal(barrier, inc=1, device_id=(nbr,),
                                device_id_type=pl.DeviceIdType.MESH)
        pl.semaphore_wait(barrier, 2)

        dA = jnp.exp(a_ref[:, :].astype(f32))
        dA1 = dA.T.astype(bf)
        dAb = jnp.broadcast_to(dA1[None], (b, n, d))

        Cb = c_ref[:, :, :].astype(bf)

        u_ref[...] = (x_ref[:, :, :].astype(bf)[:, :, None, :]
                      * b_ref[:, :, :].astype(bf)[:, :, :, None])

        def blk(i, h):
            t0 = i * BLK
            ublk = u_ref[:, pl.ds(t0, BLK)]
            hs = []
            for j in range(BLK):
                h = h * dAb + ublk[:, j]
                hs.append(h)
            u_ref[:, pl.ds(t0, BLK)] = jnp.stack(hs, axis=1)
            return h

        h_fin = lax.fori_loop(0, s // BLK, blk, jnp.zeros((b, n, d), bf))

        comm_ref[0] = h_fin
        rdma = pltpu.make_async_remote_copy(
            src_ref=comm_ref.at[0],
            dst_ref=comm_ref.at[1],
            send_sem=send_sem,
            recv_sem=recv_sem,
            device_id=(right,),
            device_id_type=pl.DeviceIdType.MESH,
        )
        rdma.start()


        glist = []
        g = dA1
        for _ in range(T_CORR):
            glist.append(g)
            g = g * dA1
        G = jnp.stack(glist, axis=0)

        out_ref[...] = jnp.sum(
            u_ref[...] * Cb[:, :, :, None], axis=2
        ).astype(f32)

        rdma.wait_recv()

        @pl.when(my != 0)
        def _():
            h_in = comm_ref[1]
            corr = jnp.sum(
                h_in[:, None] * G[None] * Cb[:, :T_CORR, :, None], axis=2
            ).astype(f32)
            out_ref[:, :T_CORR] = out_ref[:, :T_CORR] + corr

        rdma.wait_send()

    return pl.pallas_call(
        body,
        out_shape=jax.ShapeDtypeStruct((b, s, d), f32),
        in_specs=[pl.BlockSpec(memory_space=pltpu.VMEM)] * 4,
        out_specs=pl.BlockSpec(memory_space=pltpu.VMEM),
        scratch_shapes=[
            pltpu.VMEM((b, s, n, d), bf),
            pltpu.VMEM((2, b, n, d), bf),
            pltpu.SemaphoreType.DMA,
            pltpu.SemaphoreType.DMA,
        ],
        compiler_params=pltpu.CompilerParams(collective_id=0),
    )(x, A, B, C)
